# baseline (speedup 1.0000x reference)
import jax
import jax.numpy as jnp
from jax.experimental import pallas as pl
from jax.experimental.pallas import tpu as pltpu

_NUM_CLASSES = 20
_SUBLANE = 8


def _round_up(a, m):
    return (a + m - 1) // m * m


def _head_body(x1_ref, x2_ref, x3_ref, w1_ref, w2_ref, w3_ref, b_ref, o_ref):
    acc = jnp.dot(x1_ref[...], w1_ref[...], preferred_element_type=jnp.float32)
    acc = acc + jnp.dot(x2_ref[...], w2_ref[...],
                        preferred_element_type=jnp.float32)
    acc = acc + jnp.dot(x3_ref[...], w3_ref[...],
                        preferred_element_type=jnp.float32)
    acc = acc + b_ref[...]
    acc = jnp.maximum(acc, 0.0)
    o_ref[...] = acc[:, :_NUM_CLASSES]


def kernel(pooled_output, w_t_pad, b_pad):
    n, h = pooled_output.shape
    l_pad = w_t_pad.shape[1]
    ht = h // 3

    tm = min(2048, _round_up(n, _SUBLANE))
    n_pad = _round_up(n, tm)
    x = pooled_output
    if n_pad > n:
        x = jnp.pad(x, ((0, n_pad - n), (0, 0)))

    out = pl.pallas_call(
        _head_body,
        out_shape=jax.ShapeDtypeStruct((n_pad, _NUM_CLASSES), jnp.float32),
        grid=(n_pad // tm,),
        in_specs=[
            pl.BlockSpec((tm, ht), lambda i: (i, 0)),
            pl.BlockSpec((tm, ht), lambda i: (i, 1)),
            pl.BlockSpec((tm, ht), lambda i: (i, 2)),
            pl.BlockSpec((ht, l_pad), lambda i: (0, 0)),
            pl.BlockSpec((ht, l_pad), lambda i: (1, 0)),
            pl.BlockSpec((ht, l_pad), lambda i: (2, 0)),
            pl.BlockSpec((1, l_pad), lambda i: (0, 0)),
        ],
        out_specs=pl.BlockSpec((tm, _NUM_CLASSES), lambda i: (i, 0)),
        compiler_params=pltpu.CompilerParams(
            dimension_semantics=("parallel",),
        ),
    )(x, x, x, w_t_pad, w_t_pad, w_t_pad, b_pad)

    return out[:n]


# speedup vs baseline: 1.0263x; 1.0263x over previous
import jax
import jax.numpy as jnp
from jax.experimental import pallas as pl
from jax.experimental.pallas import tpu as pltpu

_NUM_CLASSES = 20
_SUBLANE = 8


def _round_up(a, m):
    return (a + m - 1) // m * m


def _head_body(x1_ref, x2_ref, w_ref, b_ref, o_ref):
    w = w_ref[...]
    b = b_ref[...]
    tm = x1_ref.shape[0]
    acc1 = jnp.dot(x1_ref[...], w, preferred_element_type=jnp.float32)
    o_ref[:tm, :] = jnp.maximum(acc1 + b, 0.0)[:, :_NUM_CLASSES]
    acc2 = jnp.dot(x2_ref[...], w, preferred_element_type=jnp.float32)
    o_ref[tm:, :] = jnp.maximum(acc2 + b, 0.0)[:, :_NUM_CLASSES]


def kernel(pooled_output, w_t_pad, b_pad):
    n, h = pooled_output.shape
    l_pad = w_t_pad.shape[1]

    tm = min(1024, _round_up(n, _SUBLANE))
    n_pad = _round_up(n, 2 * tm)
    x = pooled_output
    if n_pad > n:
        x = jnp.pad(x, ((0, n_pad - n), (0, 0)))

    out = pl.pallas_call(
        _head_body,
        out_shape=jax.ShapeDtypeStruct((n_pad, _NUM_CLASSES), jnp.float32),
        grid=(n_pad // (2 * tm),),
        in_specs=[
            pl.BlockSpec((tm, h), lambda i: (2 * i, 0)),
            pl.BlockSpec((tm, h), lambda i: (2 * i + 1, 0)),
            pl.BlockSpec((h, l_pad), lambda i: (0, 0)),
            pl.BlockSpec((1, l_pad), lambda i: (0, 0)),
        ],
        out_specs=pl.BlockSpec((2 * tm, _NUM_CLASSES), lambda i: (i, 0)),
        compiler_params=pltpu.CompilerParams(
            dimension_semantics=("parallel",),
        ),
    )(x, x, w_t_pad, b_pad)

    return out[:n]


# speedup vs baseline: 1.0272x; 1.0009x over previous
import jax
import jax.numpy as jnp
from jax.experimental import pallas as pl
from jax.experimental.pallas import tpu as pltpu

_NUM_CLASSES = 20
_SUBLANE = 8


def _round_up(a, m):
    return (a + m - 1) // m * m


def _head_body(x1_ref, x2_ref, w1_ref, w2_ref, b_ref, o_ref):
    acc = jnp.dot(x1_ref[...], w1_ref[...], preferred_element_type=jnp.float32)
    acc = acc + jnp.dot(x2_ref[...], w2_ref[...],
                        preferred_element_type=jnp.float32)
    acc = acc + b_ref[...]
    acc = jnp.maximum(acc, 0.0)
    o_ref[...] = acc[:, :_NUM_CLASSES]


def kernel(pooled_output, w_t_pad, b_pad):
    n, h = pooled_output.shape
    l_pad = w_t_pad.shape[1]
    hh = h // 2

    tm = min(2048, _round_up(n, _SUBLANE))
    n_pad = _round_up(n, tm)
    x = pooled_output
    if n_pad > n:
        x = jnp.pad(x, ((0, n_pad - n), (0, 0)))

    out = pl.pallas_call(
        _head_body,
        out_shape=jax.ShapeDtypeStruct((n_pad, _NUM_CLASSES), jnp.float32),
        grid=(n_pad // tm,),
        in_specs=[
            pl.BlockSpec((tm, hh), lambda i: (i, 0)),
            pl.BlockSpec((tm, hh), lambda i: (i, 1)),
            pl.BlockSpec((hh, l_pad), lambda i: (0, 0)),
            pl.BlockSpec((hh, l_pad), lambda i: (1, 0)),
            pl.BlockSpec((1, l_pad), lambda i: (0, 0)),
        ],
        out_specs=pl.BlockSpec((tm, _NUM_CLASSES), lambda i: (i, 0)),
        compiler_params=pltpu.CompilerParams(
            dimension_semantics=("parallel",),
        ),
    )(x, x, w_t_pad, w_t_pad, b_pad)

    return out[:n]
